# Initial kernel scaffold; baseline (speedup 1.0000x reference)
import jax
import jax.numpy as jnp
from jax.experimental import pallas as pl


def kernel(x, tables):
    raise NotImplementedError("write your pallas kernel here")


# speedup vs baseline: 19.8982x; 19.8982x over previous
import functools

import jax
import jax.numpy as jnp
import numpy as np
from jax import lax
from jax.experimental import pallas as pl
from jax.experimental.pallas import tpu as pltpu
from jax.experimental.pallas import tpu_sc as plsc

_N_LEVELS = 16
_F = 2
_LOG2_T = 19
_T = 1 << _LOG2_T
_MASK = _T - 1
_P1 = int(np.uint32(2654435761).view(np.int32))
_P2 = 805459861

_FACTOR = np.exp((np.log(512.0) - np.log(16.0)) / (_N_LEVELS - 1))
_GS = [
    float(np.float32(2.0 / float(np.floor(16.0 * _FACTOR**i))))
    for i in range(_N_LEVELS)
]

_C = 512
_W = 8 * _C


def _sc_info():
    try:
        info = plsc.get_sparse_core_info()
        return info.num_cores, info.num_subcores
    except Exception:
        return 2, 16


@functools.lru_cache(maxsize=None)
def _build(n):
    nc, ns = _sc_info()
    nw = nc * ns
    pw = n // nw
    nchunk = pw // _C
    nf = _N_LEVELS * _F
    mesh = plsc.VectorSubcoreMesh(
        core_axis_name="c", subcore_axis_name="s", num_cores=nc, num_subcores=ns
    )

    def body(x0h, x1h, x2h, tabh, outh, x0b, x1b, x2b, wb, idxb0, idxb1,
             rowsb0, rowsb1, outb, sem0, sem1):
        wid = lax.axis_index("c") * ns + lax.axis_index("s")
        iota = lax.iota(jnp.int32, 16)
        lane_nf = iota * nf
        idxbs = (idxb0, idxb1)
        rowsbs = (rowsb0, rowsb1)
        sems = (sem0, sem1)
        xbs = (x0b, x1b, x2b)

        def idx_pass(lvl, par):
            gs = _GS[lvl]
            ioff = lvl * _T

            def g_body(g, _):
                o = g * 16
                cs = []
                for d in range(3):
                    xv = xbs[d][pl.ds(o, 16)]
                    q = (xv + 1.0) / gs
                    bi = q.astype(jnp.int32)
                    bf = bi.astype(jnp.float32)
                    minv = bf * gs - 1.0
                    maxv = minv + gs
                    wb[par, d, pl.ds(o, 16)] = (xv - minv) / (maxv - minv)
                    if d == 0:
                        c, cp = bi, bi + jnp.int32(1)
                    elif d == 1:
                        c = bi * jnp.int32(_P1)
                        cp = c + jnp.int32(_P1)
                    else:
                        c = bi * jnp.int32(_P2)
                        cp = c + jnp.int32(_P2)
                    cs.append((c, cp))
                for j in range(8):
                    h = cs[0][(j >> 2) & 1] ^ cs[1][(j >> 1) & 1] ^ cs[2][j & 1]
                    hv2 = ((h & jnp.int32(_MASK)) | jnp.int32(ioff)) * 2
                    idxbs[par][pl.ds(j * _C + o, 16)] = hv2
                    idxbs[par][pl.ds(_W + j * _C + o, 16)] = hv2 + 1
                return 0

            lax.fori_loop(0, _C // 16, g_body, 0)

        def start_gather(par):
            return pltpu.async_copy(
                tabh.at[idxbs[par]], rowsbs[par], sems[par]
            )

        def interp_pass(lvl, par):
            def g_body(g, _):
                o = g * 16
                w0 = wb[par, 0, pl.ds(o, 16)]
                w1 = wb[par, 1, pl.ds(o, 16)]
                w2 = wb[par, 2, pl.ds(o, 16)]
                for f in range(2):
                    m = [
                        rowsbs[par][pl.ds(f * _W + j * _C + o, 16)]
                        for j in range(8)
                    ]
                    m = [m[2 * a] + w2 * (m[2 * a + 1] - m[2 * a])
                         for a in range(4)]
                    m = [m[2 * a] + w1 * (m[2 * a + 1] - m[2 * a])
                         for a in range(2)]
                    r = m[0] + w0 * (m[1] - m[0])
                    plsc.store_scatter(
                        outb, [lane_nf + (o * nf + 2 * lvl + f)], r
                    )
                return 0

            lax.fori_loop(0, _C // 16, g_body, 0)

        def chunk_body(c, _):
            base = wid * pw + c * _C
            pltpu.sync_copy(x0h.at[pl.ds(base, _C)], x0b)
            pltpu.sync_copy(x1h.at[pl.ds(base, _C)], x1b)
            pltpu.sync_copy(x2h.at[pl.ds(base, _C)], x2b)
            idx_pass(0, 0)
            dmas = [start_gather(0), None]
            for lvl in range(_N_LEVELS):
                par = lvl & 1
                if lvl + 1 < _N_LEVELS:
                    nxt = (lvl + 1) & 1
                    idx_pass(lvl + 1, nxt)
                    dmas[nxt] = start_gather(nxt)
                dmas[par].wait()
                interp_pass(lvl, par)
            pltpu.sync_copy(outb, outh.at[pl.ds(base * nf, _C * nf)])
            return 0

        lax.fori_loop(0, nchunk, chunk_body, 0)

    return pl.kernel(
        body,
        out_type=jax.ShapeDtypeStruct((n * nf,), jnp.float32),
        mesh=mesh,
        compiler_params=pltpu.CompilerParams(needs_layout_passes=False),
        scratch_types=[
            pltpu.VMEM((_C,), jnp.float32),
            pltpu.VMEM((_C,), jnp.float32),
            pltpu.VMEM((_C,), jnp.float32),
            pltpu.VMEM((2, 3, _C), jnp.float32),
            pltpu.VMEM((2 * _W,), jnp.int32),
            pltpu.VMEM((2 * _W,), jnp.int32),
            pltpu.VMEM((2 * _W,), jnp.float32),
            pltpu.VMEM((2 * _W,), jnp.float32),
            pltpu.VMEM((_C * nf,), jnp.float32),
            pltpu.SemaphoreType.DMA,
            pltpu.SemaphoreType.DMA,
        ],
    )


def kernel(x, tables):
    n = x.shape[0]
    x0 = x[:, 0]
    x1 = x[:, 1]
    x2 = x[:, 2]
    tab = tables.reshape(_N_LEVELS * _T * _F)
    out = _build(n)(x0, x1, x2, tab)
    return out.reshape(n, _N_LEVELS * _F)
